# baseline (speedup 1.0000x reference)
import jax
import jax.numpy as jnp
from jax import lax
from jax.experimental import pallas as pl
from jax.experimental.pallas import tpu as pltpu
from jax.experimental.pallas import tpu_sc as plsc

N = 10000
E = 320000
D = 128
HID = 8
ALPHA = 0.1

NS = 16
NPAD = 10240
RPN = NPAD // NS
EPT = 20480
E_PAD = EPT * NS
CNK = 20
CE = EPT // CNK

_mesh = plsc.VectorSubcoreMesh(
    core_axis_name="c", subcore_axis_name="s", num_cores=1, num_subcores=NS)
_sc_params = pltpu.CompilerParams(use_tc_tiling_on_sc=False)


def _rsqrt16(d):
    i = lax.bitcast_convert_type(d, jnp.int32)
    i = jnp.full((16,), 0x5F3759DF, jnp.int32) - (i >> 1)
    y = lax.bitcast_convert_type(i, jnp.float32)
    for _ in range(3):
        y = y * (1.5 - 0.5 * d * y * y)
    return y


def _zero16(buf, n):
    def zb(i, carry):
        for j in range(16):
            buf[i * 16 + j] = jnp.zeros((16,), jnp.float32)
        return carry

    lax.fori_loop(0, n // 16, zb, 0)


def _mega_body(xpdup, row8, col8, wd8, w2c, b1d, b2d,
               psum, xs1o, xs2o,
               ridx_v, cidx_v, wb0, wb1, eb0, eb1, xp_v, seg_v, q_v,
               deg_v, dinv_v, w2c_v, b1_v, b2_v, vb,
               accD, acc1,
               g0, g1, s0, s1, dsem):
    s = lax.axis_index("s")

    _zero16(eb0, RPN)

    def zd(i, carry):
        deg_v[pl.ds(16 * i, 16)] = jnp.zeros((16,), jnp.float32)
        return carry

    lax.fori_loop(0, RPN // 16, zd, 0)
    pltpu.sync_copy(eb0.at[pl.ds(0, RPN)], acc1.at[pl.ds(s * RPN, RPN)])
    pltpu.sync_copy(deg_v, accD.at[pl.ds(s * RPN, RPN)])
    pltpu.sync_copy(row8.at[pl.ds(s * CNK, CNK)], ridx_v)
    pltpu.sync_copy(col8.at[pl.ds(s * CNK, CNK)], cidx_v)
    pltpu.sync_copy(xpdup.at[pl.ds(s * RPN, RPN)], xp_v)
    pltpu.sync_copy(w2c, w2c_v)
    pltpu.sync_copy(b1d, b1_v)
    pltpu.sync_copy(b2d, b2_v)
    plsc.subcore_barrier()

    wbs = [wb0, wb1]
    for k in range(CNK):
        wb = wbs[k & 1]
        pltpu.sync_copy(wd8.at[s * CNK + k], wb)
        pltpu.async_copy(wb, accD.at[cidx_v.at[k]], dsem, add=True).wait()
    plsc.subcore_barrier()

    pltpu.sync_copy(accD.at[pl.ds(s * RPN, RPN)], deg_v)

    def node_a(i, carry):
        d = deg_v[pl.ds(16 * i, 16)] + 1.0
        y = _rsqrt16(d)
        dinv_v[pl.ds(16 * i, 16)] = y
        for j in range(16):
            r = i * 16 + j
            eb0[r] = xp_v[r] * y[j]
        return carry

    lax.fori_loop(0, RPN // 16, node_a, 0)
    pltpu.sync_copy(eb0.at[pl.ds(0, RPN)], xs1o.at[pl.ds(s * RPN, RPN)])
    plsc.subcore_barrier()

    def edge_pass(src, accT):
        ebs = [eb0, eb1]
        gss = [g0, g1]
        sss = [s0, s1]

        def mul(b, wb):
            def body(i, carry):
                wv = wb[pl.ds(16 * i, 16)]
                for j in range(16):
                    b[i * 16 + j] = b[i * 16 + j] * wv[j]
                return carry

            lax.fori_loop(0, CE // 16, body, 0, unroll=2)

        g = [None] * CNK
        sc = [None] * CNK
        g[0] = pltpu.async_copy(src.at[ridx_v.at[0]], ebs[0], gss[0])
        g[1] = pltpu.async_copy(src.at[ridx_v.at[1]], ebs[1], gss[1])
        wbs = [wb0, wb1]
        for k in range(CNK):
            p = k & 1
            pltpu.sync_copy(wd8.at[s * CNK + k], wbs[p])
            g[k].wait()
            mul(ebs[p], wbs[p])
            sc[k] = pltpu.async_copy(ebs[p], accT.at[cidx_v.at[k]],
                                     sss[p], add=True)
            if k + 2 < CNK:
                sc[k].wait()
                g[k + 2] = pltpu.async_copy(src.at[ridx_v.at[k + 2]],
                                            ebs[p], gss[p])
        sc[CNK - 2].wait()
        sc[CNK - 1].wait()

    edge_pass(xs1o, acc1)
    plsc.subcore_barrier()

    pltpu.sync_copy(acc1.at[pl.ds(s * RPN, RPN)], seg_v)
    plsc.subcore_barrier()
    _zero16(eb0, RPN)
    pltpu.sync_copy(eb0.at[pl.ds(0, RPN)], acc1.at[pl.ds(s * RPN, RPN)])
    w2rows = [w2c_v[k] for k in range(HID)]
    b1c = b1_v[...]

    def node_b(i, carry):
        dv = dinv_v[pl.ds(16 * i, 16)]
        for j in range(16):
            r = i * 16 + j
            dj = dv[j]
            xp = xp_v[r]
            agg = dj * seg_v[r] + (dj * dj) * xp
            h1 = jnp.maximum(ALPHA * xp + (1.0 - ALPHA) * agg + b1c, 0.0)
            q = h1[0] * w2rows[0]
            for kk in range(1, HID):
                q = q + h1[kk] * w2rows[kk]
            q_v[r] = q
            eb0[r] = q * dj
        return carry

    lax.fori_loop(0, RPN // 16, node_b, 0)
    pltpu.sync_copy(eb0.at[pl.ds(0, RPN)], xs2o.at[pl.ds(s * RPN, RPN)])
    plsc.subcore_barrier()

    edge_pass(xs2o, acc1)
    plsc.subcore_barrier()

    pltpu.sync_copy(acc1.at[pl.ds(s * RPN, RPN)], seg_v)
    b2c = b2_v[...]

    def node_c(i, vs):
        dv = dinv_v[pl.ds(16 * i, 16)]
        for j in range(16):
            r = i * 16 + j
            dj = dv[j]
            q = q_v[r]
            agg = dj * seg_v[r] + (dj * dj) * q
            h2 = jnp.maximum(ALPHA * q + (1.0 - ALPHA) * agg + b2c, 0.0)
            keep = (s * RPN + r) < N
            vs = vs + jnp.where(keep, h2, jnp.zeros((16,), jnp.float32))
        return vs

    vsum = lax.fori_loop(0, RPN // 16, node_c, jnp.zeros((16,), jnp.float32))
    vb[...] = vsum
    pltpu.sync_copy(vb, psum.at[s])


_mega_call = pl.kernel(
    _mega_body,
    out_type=(
        jax.ShapeDtypeStruct((NS, 16), jnp.float32),
        jax.ShapeDtypeStruct((NPAD, 16), jnp.float32),
        jax.ShapeDtypeStruct((NPAD, 16), jnp.float32),
    ),
    mesh=_mesh,
    compiler_params=_sc_params,
    scratch_types=[
        pltpu.VMEM((CNK, CE), jnp.int32),
        pltpu.VMEM((CNK, CE), jnp.int32),
        pltpu.VMEM((CE,), jnp.float32),
        pltpu.VMEM((CE,), jnp.float32),
        pltpu.VMEM((CE, 16), jnp.float32),
        pltpu.VMEM((CE, 16), jnp.float32),
        pltpu.VMEM((RPN, 16), jnp.float32),
        pltpu.VMEM((RPN, 16), jnp.float32),
        pltpu.VMEM((RPN, 16), jnp.float32),
        pltpu.VMEM((RPN,), jnp.float32),
        pltpu.VMEM((RPN,), jnp.float32),
        pltpu.VMEM((HID, 16), jnp.float32),
        pltpu.VMEM((16,), jnp.float32),
        pltpu.VMEM((16,), jnp.float32),
        pltpu.VMEM((16,), jnp.float32),
        pltpu.VMEM_SHARED((NPAD,), jnp.float32),
        pltpu.VMEM_SHARED((NPAD, 16), jnp.float32),
        pltpu.SemaphoreType.DMA,
        pltpu.SemaphoreType.DMA,
        pltpu.SemaphoreType.DMA,
        pltpu.SemaphoreType.DMA,
        pltpu.SemaphoreType.DMA,
    ],
)


def _tc_xp(x_ref, w1_ref, xp_ref):
    xp = lax.dot_general(x_ref[...], w1_ref[...], (((1,), (1,)), ((), ())),
                         preferred_element_type=jnp.float32)
    xpp = jnp.concatenate(
        [xp, jnp.zeros((NPAD - N, HID), jnp.float32)], axis=0)
    xp_ref[...] = jnp.concatenate([xpp, xpp], axis=1)


def _tc_head(ps_ref, wl1_ref, bl1_ref, wl2_ref, bl2_ref, out_ref):
    ssum = jnp.sum(ps_ref[...], axis=0)[None, :HID]
    t1 = jnp.sum(wl1_ref[...] * ssum, axis=1) + bl1_ref[...]
    hh = jnp.maximum(t1, 0.0)
    out = jnp.sum(wl2_ref[...][0] * hh) + bl2_ref[...][0]
    out_ref[...] = out.reshape(1, 1)


_tcxp_call = pl.pallas_call(
    _tc_xp,
    out_shape=jax.ShapeDtypeStruct((NPAD, 16), jnp.float32),
)

_tchead_call = pl.pallas_call(
    _tc_head,
    out_shape=jax.ShapeDtypeStruct((1, 1), jnp.float32),
)


def kernel(x, edge_index, edge_attr, W1, b1, W2, b2, Wl1, bl1, Wl2, bl2):
    row = edge_index[0]
    col = edge_index[1]
    pad = E_PAD - E
    rowp = jnp.concatenate([row, jnp.zeros((pad,), row.dtype)])
    colp = jnp.concatenate([col, jnp.zeros((pad,), col.dtype)])
    wp = jnp.concatenate([edge_attr, jnp.zeros((pad,), edge_attr.dtype)])
    row8 = rowp.reshape(NS * CNK, CE)
    col8 = colp.reshape(NS * CNK, CE)
    wd8 = wp.reshape(NS * CNK, CE)
    w2c = jnp.tile(W2.T, (1, 2))
    b1d = jnp.tile(b1, 2)
    b2d = jnp.tile(b2, 2)
    xpdup = _tcxp_call(x, W1)
    psum, _, _ = _mega_call(xpdup, row8, col8, wd8, w2c, b1d, b2d)
    out = _tchead_call(psum, Wl1, bl1, Wl2, bl2)
    return out.reshape(1)


# speedup vs baseline: 1.0275x; 1.0275x over previous
import jax
import jax.numpy as jnp
from jax import lax
from jax.experimental import pallas as pl
from jax.experimental.pallas import tpu as pltpu
from jax.experimental.pallas import tpu_sc as plsc

N = 10000
E = 320000
D = 128
HID = 8
ALPHA = 0.1

NS = 16
NPAD = 10240
RPN = NPAD // NS
EPT = 20480
E_PAD = EPT * NS
CNK = 20
CE = EPT // CNK

_mesh = plsc.VectorSubcoreMesh(
    core_axis_name="c", subcore_axis_name="s", num_cores=1, num_subcores=NS)
_sc_params = pltpu.CompilerParams(use_tc_tiling_on_sc=False)


def _rsqrt16(d):
    i = lax.bitcast_convert_type(d, jnp.int32)
    i = jnp.full((16,), 0x5F3759DF, jnp.int32) - (i >> 1)
    y = lax.bitcast_convert_type(i, jnp.float32)
    for _ in range(3):
        y = y * (1.5 - 0.5 * d * y * y)
    return y


def _zero16(buf, n):
    def zb(i, carry):
        for j in range(16):
            buf[i * 16 + j] = jnp.zeros((16,), jnp.float32)
        return carry

    lax.fori_loop(0, n // 16, zb, 0)


def _mega_body(xpdup, row8, col8, wd8, w2c, b1d, b2d,
               psum, xs1o, xs2o,
               ridx_v, cidx_v, wb0, wb1, eb0, eb1, xp_v, seg_v, q_v,
               deg_v, dinv_v, w2c_v, b1_v, b2_v, vb,
               accD, acc1,
               g0, g1, s0, s1, dsem, dsem2):
    s = lax.axis_index("s")

    _zero16(eb0, RPN)

    def zd(i, carry):
        deg_v[pl.ds(16 * i, 16)] = jnp.zeros((16,), jnp.float32)
        return carry

    lax.fori_loop(0, RPN // 16, zd, 0)
    pltpu.sync_copy(eb0.at[pl.ds(0, RPN)], acc1.at[pl.ds(s * RPN, RPN)])
    pltpu.sync_copy(deg_v, accD.at[pl.ds(s * RPN, RPN)])
    pltpu.sync_copy(row8.at[pl.ds(s * CNK, CNK)], ridx_v)
    pltpu.sync_copy(col8.at[pl.ds(s * CNK, CNK)], cidx_v)
    pltpu.sync_copy(xpdup.at[pl.ds(s * RPN, RPN)], xp_v)
    pltpu.sync_copy(w2c, w2c_v)
    pltpu.sync_copy(b1d, b1_v)
    pltpu.sync_copy(b2d, b2_v)
    plsc.subcore_barrier()

    wbs = [wb0, wb1]
    dsems = [dsem, dsem2]
    dsc = [None] * CNK
    for k in range(CNK):
        p = k & 1
        if k >= 2:
            dsc[k - 2].wait()
        pltpu.sync_copy(wd8.at[s * CNK + k], wbs[p])
        dsc[k] = pltpu.async_copy(wbs[p], accD.at[cidx_v.at[k]],
                                  dsems[p], add=True)
    dsc[CNK - 2].wait()
    dsc[CNK - 1].wait()
    plsc.subcore_barrier()

    pltpu.sync_copy(accD.at[pl.ds(s * RPN, RPN)], deg_v)

    def node_a(i, carry):
        d = deg_v[pl.ds(16 * i, 16)] + 1.0
        y = _rsqrt16(d)
        dinv_v[pl.ds(16 * i, 16)] = y
        for j in range(16):
            r = i * 16 + j
            eb0[r] = xp_v[r] * y[j]
        return carry

    lax.fori_loop(0, RPN // 16, node_a, 0)
    pltpu.sync_copy(eb0.at[pl.ds(0, RPN)], xs1o.at[pl.ds(s * RPN, RPN)])
    plsc.subcore_barrier()

    def edge_pass(src, accT):
        ebs = [eb0, eb1]
        gss = [g0, g1]
        sss = [s0, s1]

        def mul(b, wb):
            def body(i, carry):
                wv = wb[pl.ds(16 * i, 16)]
                for j in range(16):
                    b[i * 16 + j] = b[i * 16 + j] * wv[j]
                return carry

            lax.fori_loop(0, CE // 16, body, 0, unroll=2)

        g = [None] * CNK
        sc = [None] * CNK
        g[0] = pltpu.async_copy(src.at[ridx_v.at[0]], ebs[0], gss[0])
        g[1] = pltpu.async_copy(src.at[ridx_v.at[1]], ebs[1], gss[1])
        wbs = [wb0, wb1]
        for k in range(CNK):
            p = k & 1
            pltpu.sync_copy(wd8.at[s * CNK + k], wbs[p])
            g[k].wait()
            mul(ebs[p], wbs[p])
            sc[k] = pltpu.async_copy(ebs[p], accT.at[cidx_v.at[k]],
                                     sss[p], add=True)
            if k + 2 < CNK:
                sc[k].wait()
                g[k + 2] = pltpu.async_copy(src.at[ridx_v.at[k + 2]],
                                            ebs[p], gss[p])
        sc[CNK - 2].wait()
        sc[CNK - 1].wait()

    edge_pass(xs1o, acc1)
    plsc.subcore_barrier()

    pltpu.sync_copy(acc1.at[pl.ds(s * RPN, RPN)], seg_v)
    plsc.subcore_barrier()
    _zero16(eb0, RPN)
    pltpu.sync_copy(eb0.at[pl.ds(0, RPN)], acc1.at[pl.ds(s * RPN, RPN)])
    w2rows = [w2c_v[k] for k in range(HID)]
    b1c = b1_v[...]

    def node_b(i, carry):
        dv = dinv_v[pl.ds(16 * i, 16)]
        for j in range(16):
            r = i * 16 + j
            dj = dv[j]
            xp = xp_v[r]
            agg = dj * seg_v[r] + (dj * dj) * xp
            h1 = jnp.maximum(ALPHA * xp + (1.0 - ALPHA) * agg + b1c, 0.0)
            q = h1[0] * w2rows[0]
            for kk in range(1, HID):
                q = q + h1[kk] * w2rows[kk]
            q_v[r] = q
            eb0[r] = q * dj
        return carry

    lax.fori_loop(0, RPN // 16, node_b, 0)
    pltpu.sync_copy(eb0.at[pl.ds(0, RPN)], xs2o.at[pl.ds(s * RPN, RPN)])
    plsc.subcore_barrier()

    edge_pass(xs2o, acc1)
    plsc.subcore_barrier()

    pltpu.sync_copy(acc1.at[pl.ds(s * RPN, RPN)], seg_v)
    b2c = b2_v[...]

    def node_c(i, vs):
        dv = dinv_v[pl.ds(16 * i, 16)]
        for j in range(16):
            r = i * 16 + j
            dj = dv[j]
            q = q_v[r]
            agg = dj * seg_v[r] + (dj * dj) * q
            h2 = jnp.maximum(ALPHA * q + (1.0 - ALPHA) * agg + b2c, 0.0)
            keep = (s * RPN + r) < N
            vs = vs + jnp.where(keep, h2, jnp.zeros((16,), jnp.float32))
        return vs

    vsum = lax.fori_loop(0, RPN // 16, node_c, jnp.zeros((16,), jnp.float32))
    vb[...] = vsum
    pltpu.sync_copy(vb, psum.at[s])


_mega_call = pl.kernel(
    _mega_body,
    out_type=(
        jax.ShapeDtypeStruct((NS, 16), jnp.float32),
        jax.ShapeDtypeStruct((NPAD, 16), jnp.float32),
        jax.ShapeDtypeStruct((NPAD, 16), jnp.float32),
    ),
    mesh=_mesh,
    compiler_params=_sc_params,
    scratch_types=[
        pltpu.VMEM((CNK, CE), jnp.int32),
        pltpu.VMEM((CNK, CE), jnp.int32),
        pltpu.VMEM((CE,), jnp.float32),
        pltpu.VMEM((CE,), jnp.float32),
        pltpu.VMEM((CE, 16), jnp.float32),
        pltpu.VMEM((CE, 16), jnp.float32),
        pltpu.VMEM((RPN, 16), jnp.float32),
        pltpu.VMEM((RPN, 16), jnp.float32),
        pltpu.VMEM((RPN, 16), jnp.float32),
        pltpu.VMEM((RPN,), jnp.float32),
        pltpu.VMEM((RPN,), jnp.float32),
        pltpu.VMEM((HID, 16), jnp.float32),
        pltpu.VMEM((16,), jnp.float32),
        pltpu.VMEM((16,), jnp.float32),
        pltpu.VMEM((16,), jnp.float32),
        pltpu.VMEM_SHARED((NPAD,), jnp.float32),
        pltpu.VMEM_SHARED((NPAD, 16), jnp.float32),
        pltpu.SemaphoreType.DMA,
        pltpu.SemaphoreType.DMA,
        pltpu.SemaphoreType.DMA,
        pltpu.SemaphoreType.DMA,
        pltpu.SemaphoreType.DMA,
        pltpu.SemaphoreType.DMA,
    ],
)


def _tc_xp(x_ref, w1_ref, xp_ref):
    xp = lax.dot_general(x_ref[...], w1_ref[...], (((1,), (1,)), ((), ())),
                         preferred_element_type=jnp.float32)
    xpp = jnp.concatenate(
        [xp, jnp.zeros((NPAD - N, HID), jnp.float32)], axis=0)
    xp_ref[...] = jnp.concatenate([xpp, xpp], axis=1)


def _tc_head(ps_ref, wl1_ref, bl1_ref, wl2_ref, bl2_ref, out_ref):
    ssum = jnp.sum(ps_ref[...], axis=0)[None, :HID]
    t1 = jnp.sum(wl1_ref[...] * ssum, axis=1) + bl1_ref[...]
    hh = jnp.maximum(t1, 0.0)
    out = jnp.sum(wl2_ref[...][0] * hh) + bl2_ref[...][0]
    out_ref[...] = out.reshape(1, 1)


_tcxp_call = pl.pallas_call(
    _tc_xp,
    out_shape=jax.ShapeDtypeStruct((NPAD, 16), jnp.float32),
)

_tchead_call = pl.pallas_call(
    _tc_head,
    out_shape=jax.ShapeDtypeStruct((1, 1), jnp.float32),
)


def kernel(x, edge_index, edge_attr, W1, b1, W2, b2, Wl1, bl1, Wl2, bl2):
    row = edge_index[0]
    col = edge_index[1]
    pad = E_PAD - E
    rowp = jnp.concatenate([row, jnp.zeros((pad,), row.dtype)])
    colp = jnp.concatenate([col, jnp.zeros((pad,), col.dtype)])
    wp = jnp.concatenate([edge_attr, jnp.zeros((pad,), edge_attr.dtype)])
    row8 = rowp.reshape(NS * CNK, CE)
    col8 = colp.reshape(NS * CNK, CE)
    wd8 = wp.reshape(NS * CNK, CE)
    w2c = jnp.tile(W2.T, (1, 2))
    b1d = jnp.tile(b1, 2)
    b2d = jnp.tile(b2, 2)
    xpdup = _tcxp_call(x, W1)
    psum, _, _ = _mega_call(xpdup, row8, col8, wd8, w2c, b1d, b2d)
    out = _tchead_call(psum, Wl1, bl1, Wl2, bl2)
    return out.reshape(1)


# speedup vs baseline: 1.0600x; 1.0317x over previous
import jax
import jax.numpy as jnp
from jax import lax
from jax.experimental import pallas as pl
from jax.experimental.pallas import tpu as pltpu
from jax.experimental.pallas import tpu_sc as plsc

N = 10000
E = 320000
D = 128
HID = 8
ALPHA = 0.1

NS = 16
NPAD = 10240
RPN = NPAD // NS
EPT = 20480
E_PAD = EPT * NS
CNK = 20
CE = EPT // CNK

_mesh = plsc.VectorSubcoreMesh(
    core_axis_name="c", subcore_axis_name="s", num_cores=1, num_subcores=NS)
_sc_params = pltpu.CompilerParams(use_tc_tiling_on_sc=False)


def _rsqrt16(d):
    i = lax.bitcast_convert_type(d, jnp.int32)
    i = jnp.full((16,), 0x5F3759DF, jnp.int32) - (i >> 1)
    y = lax.bitcast_convert_type(i, jnp.float32)
    for _ in range(3):
        y = y * (1.5 - 0.5 * d * y * y)
    return y


def _zero16(buf, n):
    def zb(i, carry):
        for j in range(16):
            buf[i * 16 + j] = jnp.zeros((16,), jnp.float32)
        return carry

    lax.fori_loop(0, n // 16, zb, 0)


def _mega_body(xpdup, row8, col8, wd8, w2c, b1d, b2d,
               psum, xs1o, xs2o,
               ridx_v, cidx_v, wb0, wb1, eb0, eb1, xp_v, seg_v, q_v,
               deg_v, dinv_v, w2c_v, b1_v, b2_v, vb,
               accD, acc1,
               g0, g1, s0, s1, dsem, dsem2):
    s = lax.axis_index("s")

    _zero16(eb0, RPN)

    def zd(i, carry):
        deg_v[pl.ds(16 * i, 16)] = jnp.zeros((16,), jnp.float32)
        return carry

    lax.fori_loop(0, RPN // 16, zd, 0)
    pltpu.sync_copy(eb0.at[pl.ds(0, RPN)], acc1.at[pl.ds(s * RPN, RPN)])
    pltpu.sync_copy(deg_v, accD.at[pl.ds(s * RPN, RPN)])
    pltpu.sync_copy(row8.at[pl.ds(s * CNK, CNK)], ridx_v)
    pltpu.sync_copy(col8.at[pl.ds(s * CNK, CNK)], cidx_v)
    pltpu.sync_copy(xpdup.at[pl.ds(s * RPN, RPN)], xp_v)
    pltpu.sync_copy(w2c, w2c_v)
    pltpu.sync_copy(b1d, b1_v)
    pltpu.sync_copy(b2d, b2_v)
    plsc.subcore_barrier()

    wbs = [wb0, wb1]
    dsems = [dsem, dsem2]
    dsc = [None] * CNK
    for k in range(CNK):
        p = k & 1
        if k >= 2:
            dsc[k - 2].wait()
        pltpu.sync_copy(wd8.at[s * CNK + k], wbs[p])
        dsc[k] = pltpu.async_copy(wbs[p], accD.at[cidx_v.at[k]],
                                  dsems[p], add=True)
    dsc[CNK - 2].wait()
    dsc[CNK - 1].wait()
    plsc.subcore_barrier()

    pltpu.sync_copy(accD.at[pl.ds(s * RPN, RPN)], deg_v)

    def node_a(i, carry):
        d = deg_v[pl.ds(16 * i, 16)] + 1.0
        y = _rsqrt16(d)
        dinv_v[pl.ds(16 * i, 16)] = y
        for j in range(16):
            r = i * 16 + j
            eb0[r] = xp_v[r] * y[j]
        return carry

    lax.fori_loop(0, RPN // 16, node_a, 0)
    pltpu.sync_copy(eb0.at[pl.ds(0, RPN)], xs1o.at[pl.ds(s * RPN, RPN)])
    plsc.subcore_barrier()

    def edge_pass(src, accT):
        ebs = [eb0, eb1]
        gss = [g0, g1]
        sss = [s0, s1]

        def mul(b, wb):
            def body(i, carry):
                wv = wb[pl.ds(16 * i, 16)]
                for j in range(16):
                    b[i * 16 + j] = b[i * 16 + j] * wv[j]
                return carry

            lax.fori_loop(0, CE // 16, body, 0, unroll=2)

        g = [None] * CNK
        sc = [None] * CNK
        wl = [None] * CNK
        wbs = [wb0, wb1]
        wss = [dsem, dsem2]
        g[0] = pltpu.async_copy(src.at[ridx_v.at[0]], ebs[0], gss[0])
        g[1] = pltpu.async_copy(src.at[ridx_v.at[1]], ebs[1], gss[1])
        wl[0] = pltpu.async_copy(wd8.at[s * CNK + 0], wbs[0], wss[0])
        wl[1] = pltpu.async_copy(wd8.at[s * CNK + 1], wbs[1], wss[1])
        for k in range(CNK):
            p = k & 1
            g[k].wait()
            wl[k].wait()
            mul(ebs[p], wbs[p])
            sc[k] = pltpu.async_copy(ebs[p], accT.at[cidx_v.at[k]],
                                     sss[p], add=True)
            if k + 2 < CNK:
                sc[k].wait()
                g[k + 2] = pltpu.async_copy(src.at[ridx_v.at[k + 2]],
                                            ebs[p], gss[p])
                wl[k + 2] = pltpu.async_copy(wd8.at[s * CNK + k + 2],
                                             wbs[p], wss[p])
        sc[CNK - 2].wait()
        sc[CNK - 1].wait()

    edge_pass(xs1o, acc1)
    plsc.subcore_barrier()

    pltpu.sync_copy(acc1.at[pl.ds(s * RPN, RPN)], seg_v)
    plsc.subcore_barrier()
    _zero16(eb0, RPN)
    pltpu.sync_copy(eb0.at[pl.ds(0, RPN)], acc1.at[pl.ds(s * RPN, RPN)])
    w2rows = [w2c_v[k] for k in range(HID)]
    b1c = b1_v[...]

    def node_b(i, carry):
        dv = dinv_v[pl.ds(16 * i, 16)]
        for j in range(16):
            r = i * 16 + j
            dj = dv[j]
            xp = xp_v[r]
            agg = dj * seg_v[r] + (dj * dj) * xp
            h1 = jnp.maximum(ALPHA * xp + (1.0 - ALPHA) * agg + b1c, 0.0)
            q = h1[0] * w2rows[0]
            for kk in range(1, HID):
                q = q + h1[kk] * w2rows[kk]
            q_v[r] = q
            eb0[r] = q * dj
        return carry

    lax.fori_loop(0, RPN // 16, node_b, 0)
    pltpu.sync_copy(eb0.at[pl.ds(0, RPN)], xs2o.at[pl.ds(s * RPN, RPN)])
    plsc.subcore_barrier()

    edge_pass(xs2o, acc1)
    plsc.subcore_barrier()

    pltpu.sync_copy(acc1.at[pl.ds(s * RPN, RPN)], seg_v)
    b2c = b2_v[...]

    def node_c(i, vs):
        dv = dinv_v[pl.ds(16 * i, 16)]
        for j in range(16):
            r = i * 16 + j
            dj = dv[j]
            q = q_v[r]
            agg = dj * seg_v[r] + (dj * dj) * q
            h2 = jnp.maximum(ALPHA * q + (1.0 - ALPHA) * agg + b2c, 0.0)
            keep = (s * RPN + r) < N
            vs = vs + jnp.where(keep, h2, jnp.zeros((16,), jnp.float32))
        return vs

    vsum = lax.fori_loop(0, RPN // 16, node_c, jnp.zeros((16,), jnp.float32))
    vb[...] = vsum
    pltpu.sync_copy(vb, psum.at[s])


_mega_call = pl.kernel(
    _mega_body,
    out_type=(
        jax.ShapeDtypeStruct((NS, 16), jnp.float32),
        jax.ShapeDtypeStruct((NPAD, 16), jnp.float32),
        jax.ShapeDtypeStruct((NPAD, 16), jnp.float32),
    ),
    mesh=_mesh,
    compiler_params=_sc_params,
    scratch_types=[
        pltpu.VMEM((CNK, CE), jnp.int32),
        pltpu.VMEM((CNK, CE), jnp.int32),
        pltpu.VMEM((CE,), jnp.float32),
        pltpu.VMEM((CE,), jnp.float32),
        pltpu.VMEM((CE, 16), jnp.float32),
        pltpu.VMEM((CE, 16), jnp.float32),
        pltpu.VMEM((RPN, 16), jnp.float32),
        pltpu.VMEM((RPN, 16), jnp.float32),
        pltpu.VMEM((RPN, 16), jnp.float32),
        pltpu.VMEM((RPN,), jnp.float32),
        pltpu.VMEM((RPN,), jnp.float32),
        pltpu.VMEM((HID, 16), jnp.float32),
        pltpu.VMEM((16,), jnp.float32),
        pltpu.VMEM((16,), jnp.float32),
        pltpu.VMEM((16,), jnp.float32),
        pltpu.VMEM_SHARED((NPAD,), jnp.float32),
        pltpu.VMEM_SHARED((NPAD, 16), jnp.float32),
        pltpu.SemaphoreType.DMA,
        pltpu.SemaphoreType.DMA,
        pltpu.SemaphoreType.DMA,
        pltpu.SemaphoreType.DMA,
        pltpu.SemaphoreType.DMA,
        pltpu.SemaphoreType.DMA,
    ],
)


def _tc_xp(x_ref, w1_ref, xp_ref):
    xp = lax.dot_general(x_ref[...], w1_ref[...], (((1,), (1,)), ((), ())),
                         preferred_element_type=jnp.float32)
    xpp = jnp.concatenate(
        [xp, jnp.zeros((NPAD - N, HID), jnp.float32)], axis=0)
    xp_ref[...] = jnp.concatenate([xpp, xpp], axis=1)


def _tc_head(ps_ref, wl1_ref, bl1_ref, wl2_ref, bl2_ref, out_ref):
    ssum = jnp.sum(ps_ref[...], axis=0)[None, :HID]
    t1 = jnp.sum(wl1_ref[...] * ssum, axis=1) + bl1_ref[...]
    hh = jnp.maximum(t1, 0.0)
    out = jnp.sum(wl2_ref[...][0] * hh) + bl2_ref[...][0]
    out_ref[...] = out.reshape(1, 1)


_tcxp_call = pl.pallas_call(
    _tc_xp,
    out_shape=jax.ShapeDtypeStruct((NPAD, 16), jnp.float32),
)

_tchead_call = pl.pallas_call(
    _tc_head,
    out_shape=jax.ShapeDtypeStruct((1, 1), jnp.float32),
)


def kernel(x, edge_index, edge_attr, W1, b1, W2, b2, Wl1, bl1, Wl2, bl2):
    row = edge_index[0]
    col = edge_index[1]
    pad = E_PAD - E
    rowp = jnp.concatenate([row, jnp.zeros((pad,), row.dtype)])
    colp = jnp.concatenate([col, jnp.zeros((pad,), col.dtype)])
    wp = jnp.concatenate([edge_attr, jnp.zeros((pad,), edge_attr.dtype)])
    row8 = rowp.reshape(NS * CNK, CE)
    col8 = colp.reshape(NS * CNK, CE)
    wd8 = wp.reshape(NS * CNK, CE)
    w2c = jnp.tile(W2.T, (1, 2))
    b1d = jnp.tile(b1, 2)
    b2d = jnp.tile(b2, 2)
    xpdup = _tcxp_call(x, W1)
    psum, _, _ = _mega_call(xpdup, row8, col8, wd8, w2c, b1d, b2d)
    out = _tchead_call(psum, Wl1, bl1, Wl2, bl2)
    return out.reshape(1)
